# baseline (speedup 1.0000x reference)
import jax
import jax.numpy as jnp
from jax.experimental import pallas as pl
from jax.experimental.pallas import tpu as pltpu

TOKENS = 8192
HIDDEN = 2048
EXPERTS = 16
BT = 256


def _body(x_ref, wt_ref, logits_ref, idx_ref):
    xb = x_ref[...]
    wt = wt_ref[...]
    l = jnp.dot(xb, wt, preferred_element_type=jnp.float32)
    logits_ref[...] = l
    m = jnp.max(l, axis=-1, keepdims=True)
    e_iota = jax.lax.broadcasted_iota(jnp.int32, (BT, EXPERTS), 1)
    idx = jnp.min(jnp.where(l == m, e_iota, EXPERTS), axis=-1, keepdims=True)
    idx_ref[...] = idx


def kernel(x, W):
    wt = W.T
    logits, idx = pl.pallas_call(
        _body,
        grid=(TOKENS // BT,),
        in_specs=[
            pl.BlockSpec((BT, HIDDEN), lambda i: (i, 0)),
            pl.BlockSpec((HIDDEN, EXPERTS), lambda i: (0, 0)),
        ],
        out_specs=[
            pl.BlockSpec((BT, EXPERTS), lambda i: (i, 0)),
            pl.BlockSpec((BT, 1), lambda i: (i, 0)),
        ],
        out_shape=[
            jax.ShapeDtypeStruct((TOKENS, EXPERTS), jnp.float32),
            jax.ShapeDtypeStruct((TOKENS, 1), jnp.int32),
        ],
        compiler_params=pltpu.CompilerParams(
            dimension_semantics=("arbitrary",),
        ),
    )(x, wt)
    return (logits, idx.reshape(TOKENS))


# speedup vs baseline: 1.4622x; 1.4622x over previous
import jax
import jax.numpy as jnp
from jax.experimental import pallas as pl
from jax.experimental.pallas import tpu as pltpu

TOKENS = 8192
HIDDEN = 2048
EXPERTS = 16
BT = 1024


def _body(x_ref, wt_ref, logits_ref, idx_ref):
    xb = x_ref[...]
    wt = wt_ref[...]
    l = jnp.dot(xb, wt, preferred_element_type=jnp.float32)
    logits_ref[...] = l
    m = jnp.max(l, axis=-1, keepdims=True)
    e_iota = jax.lax.broadcasted_iota(jnp.int32, (BT, EXPERTS), 1)
    idx = jnp.min(jnp.where(l == m, e_iota, EXPERTS), axis=-1, keepdims=True)
    idx_ref[...] = idx


def kernel(x, W):
    wt = W.T
    logits, idx = pl.pallas_call(
        _body,
        grid=(TOKENS // BT,),
        in_specs=[
            pl.BlockSpec((BT, HIDDEN), lambda i: (i, 0)),
            pl.BlockSpec((HIDDEN, EXPERTS), lambda i: (0, 0)),
        ],
        out_specs=[
            pl.BlockSpec((BT, EXPERTS), lambda i: (i, 0)),
            pl.BlockSpec((BT, 1), lambda i: (i, 0)),
        ],
        out_shape=[
            jax.ShapeDtypeStruct((TOKENS, EXPERTS), jnp.float32),
            jax.ShapeDtypeStruct((TOKENS, 1), jnp.int32),
        ],
        compiler_params=pltpu.CompilerParams(
            dimension_semantics=("arbitrary",),
        ),
    )(x, wt)
    return (logits, idx.reshape(TOKENS))
